# Initial kernel scaffold; baseline (speedup 1.0000x reference)
import functools

import jax
import jax.numpy as jnp
from jax import lax
from jax.experimental import pallas as pl
from jax.experimental.pallas import tpu as pltpu
from jax.experimental.pallas import tpu_sc as plsc

N = 10000
E = 320000
C = 128

NC = 2
NS = 16
NW = NC * NS
EPW = E // NW
CHUNK = 80
NCHUNK = EPW // CHUNK
RPW = N // NS
LANES = 16
DEG_STEPS = EPW // LANES

_mesh = plsc.VectorSubcoreMesh(core_axis_name="c", subcore_axis_name="s")


@functools.partial(
    pl.kernel,
    out_type=jax.ShapeDtypeStruct((NW, N), jnp.float32),
    mesh=_mesh,
    scratch_types=[
        pltpu.VMEM((EPW,), jnp.int32),
        pltpu.VMEM((N,), jnp.float32),
    ],
)
def _deg_kernel(dst_hbm, out_hbm, dst_v, deg_v):
    wid = lax.axis_index("c") * NS + lax.axis_index("s")
    pltpu.sync_copy(dst_hbm.at[wid], dst_v)

    zeros16 = jnp.zeros((LANES,), jnp.float32)
    ones16 = jnp.ones((LANES,), jnp.float32)

    def _zero(i, carry):
        deg_v[pl.ds(i * LANES, LANES)] = zeros16
        return carry

    lax.fori_loop(0, N // LANES, _zero, 0, unroll=8)

    def _count(i, carry):
        idx = dst_v[pl.ds(i * LANES, LANES)]
        plsc.addupdate_scatter(deg_v, [idx], ones16)
        return carry

    lax.fori_loop(0, DEG_STEPS, _count, 0, unroll=4)
    pltpu.sync_copy(deg_v, out_hbm.at[wid])


def _prep_body(deg_ref, x_ref, w_ref, g_ref, dis_ref):
    deg = jnp.sum(deg_ref[...], axis=0) + 1.0
    dis = lax.rsqrt(deg)
    h = jnp.dot(x_ref[...], w_ref[...], preferred_element_type=jnp.float32)
    g_ref[...] = h * dis[:, None]
    dis_ref[...] = dis[:, None]


_BR = 1250


def _tc_prep(deg_parts, x, W):
    return pl.pallas_call(
        _prep_body,
        grid=(N // _BR,),
        in_specs=[
            pl.BlockSpec((NW, _BR), lambda i: (0, i)),
            pl.BlockSpec((_BR, C), lambda i: (i, 0)),
            pl.BlockSpec((C, C), lambda i: (0, 0)),
        ],
        out_specs=[
            pl.BlockSpec((_BR, C), lambda i: (i, 0)),
            pl.BlockSpec((_BR, 1), lambda i: (i, 0)),
        ],
        out_shape=[
            jax.ShapeDtypeStruct((N, C), jnp.float32),
            jax.ShapeDtypeStruct((N, 1), jnp.float32),
        ],
    )(deg_parts, x, W)


@functools.partial(
    pl.kernel,
    out_type=jax.ShapeDtypeStruct((NC, N, C), jnp.float32),
    mesh=_mesh,
    scratch_types=[
        pltpu.VMEM((NCHUNK, CHUNK), jnp.int32),
        pltpu.VMEM((NCHUNK, CHUNK), jnp.int32),
        pltpu.VMEM((CHUNK, C), jnp.float32),
        pltpu.VMEM_SHARED((N, C), jnp.float32),
        pltpu.SemaphoreType.DMA,
    ],
)
def _scatter_kernel(g_hbm, src_hbm, dst_hbm, zero_hbm, out_hbm,
                    src_v, dst_v, rows_v, acc_sh, sem):
    cid = lax.axis_index("c")
    sid = lax.axis_index("s")
    wid = cid * NS + sid

    stripe = pl.ds(sid * RPW, RPW)
    pltpu.sync_copy(zero_hbm.at[stripe], acc_sh.at[stripe])
    pltpu.sync_copy(src_hbm.at[wid], src_v)
    pltpu.sync_copy(dst_hbm.at[wid], dst_v)
    plsc.subcore_barrier()

    def _edge_chunk(j, carry):
        pltpu.async_copy(g_hbm.at[src_v.at[j]], rows_v, sem).wait()
        pltpu.sync_copy(rows_v, acc_sh.at[dst_v.at[j]], add=True)
        return carry

    lax.fori_loop(0, NCHUNK, _edge_chunk, 0)
    plsc.subcore_barrier()
    pltpu.sync_copy(acc_sh.at[stripe], out_hbm.at[cid].at[stripe])


def _final_body(sp_ref, g_ref, dis_ref, b_ref, o_ref):
    z = (sp_ref[0] + sp_ref[1] + g_ref[...]) * dis_ref[...] + b_ref[...]
    m = jnp.max(z, axis=1, keepdims=True)
    e = z - m
    o_ref[...] = e - jnp.log(jnp.sum(jnp.exp(e), axis=1, keepdims=True))


def _tc_final(s_parts, g, dis, b2d):
    return pl.pallas_call(
        _final_body,
        grid=(N // _BR,),
        in_specs=[
            pl.BlockSpec((NC, _BR, C), lambda i: (0, i, 0)),
            pl.BlockSpec((_BR, C), lambda i: (i, 0)),
            pl.BlockSpec((_BR, 1), lambda i: (i, 0)),
            pl.BlockSpec((1, C), lambda i: (0, 0)),
        ],
        out_specs=pl.BlockSpec((_BR, C), lambda i: (i, 0)),
        out_shape=jax.ShapeDtypeStruct((N, C), jnp.float32),
    )(s_parts, g, dis, b2d)


def kernel(x, edge_index, W, b):
    src = edge_index[0].reshape(NW, NCHUNK, CHUNK)
    dst = edge_index[1]
    deg_parts = _deg_kernel(dst.reshape(NW, EPW))
    g, dis = _tc_prep(deg_parts, x, W)
    zeros = jnp.zeros((N, C), jnp.float32)
    s_parts = _scatter_kernel(g, src, dst.reshape(NW, NCHUNK, CHUNK), zeros)
    return _tc_final(s_parts, g, dis, b.reshape(1, C))


# speedup vs baseline: 28.3058x; 28.3058x over previous
import functools

import jax
import jax.numpy as jnp
from jax import lax
from jax.experimental import pallas as pl
from jax.experimental.pallas import tpu as pltpu
from jax.experimental.pallas import tpu_sc as plsc

N = 10000
E = 320000
C = 128

NC = 2
NS = 16
NW = NC * NS
EPW = E // NW
CHUNK = 80
NCHUNK = EPW // CHUNK
RPW = 624
RTAIL = N - NS * RPW
LANES = 16
DEG_STEPS = EPW // LANES

_mesh = plsc.VectorSubcoreMesh(core_axis_name="c", subcore_axis_name="s")


@functools.partial(
    pl.kernel,
    out_type=jax.ShapeDtypeStruct((NW, N), jnp.float32),
    mesh=_mesh,
    compiler_params=pltpu.CompilerParams(needs_layout_passes=False),
    scratch_types=[
        pltpu.VMEM((EPW,), jnp.int32),
        pltpu.VMEM((N,), jnp.float32),
    ],
)
def _deg_kernel(dst_hbm, out_hbm, dst_v, deg_v):
    wid = lax.axis_index("c") * NS + lax.axis_index("s")
    pltpu.sync_copy(dst_hbm.at[wid], dst_v)

    zeros16 = jnp.zeros((LANES,), jnp.float32)
    ones16 = jnp.ones((LANES,), jnp.float32)

    def _zero(i, carry):
        deg_v[pl.ds(i * LANES, LANES)] = zeros16
        return carry

    lax.fori_loop(0, N // LANES, _zero, 0, unroll=8)

    def _count(i, carry):
        idx = dst_v[pl.ds(i * LANES, LANES)]
        plsc.addupdate_scatter(deg_v, [idx], ones16)
        return carry

    lax.fori_loop(0, DEG_STEPS, _count, 0, unroll=4)
    pltpu.sync_copy(deg_v, out_hbm.at[wid])


def _prep_body(deg_ref, x_ref, w_ref, g_ref, dis_ref):
    deg = jnp.sum(deg_ref[...], axis=1) + 1.0
    dis = lax.rsqrt(deg)
    h = jnp.dot(x_ref[...], w_ref[...], preferred_element_type=jnp.float32)
    g_ref[...] = h * dis[:, None]
    dis_ref[...] = dis[:, None]


_BR = 2000


def _tc_prep(deg_parts, x, W):
    return pl.pallas_call(
        _prep_body,
        grid=(N // _BR,),
        in_specs=[
            pl.BlockSpec((_BR, NW), lambda i: (i, 0)),
            pl.BlockSpec((_BR, C), lambda i: (i, 0)),
            pl.BlockSpec((C, C), lambda i: (0, 0)),
        ],
        out_specs=[
            pl.BlockSpec((_BR, C), lambda i: (i, 0)),
            pl.BlockSpec((_BR, 1), lambda i: (i, 0)),
        ],
        out_shape=[
            jax.ShapeDtypeStruct((N, C), jnp.float32),
            jax.ShapeDtypeStruct((N, 1), jnp.float32),
        ],
    )(deg_parts, x, W)


@functools.partial(
    pl.kernel,
    out_type=jax.ShapeDtypeStruct((NC, N, C), jnp.float32),
    mesh=_mesh,
    compiler_params=pltpu.CompilerParams(needs_layout_passes=False),
    scratch_types=[
        pltpu.VMEM((NCHUNK, CHUNK), jnp.int32),
        pltpu.VMEM((NCHUNK, CHUNK), jnp.int32),
        pltpu.VMEM((CHUNK, C), jnp.float32),
        pltpu.VMEM_SHARED((N, C), jnp.float32),
        pltpu.SemaphoreType.DMA,
    ],
)
def _scatter_kernel(g_hbm, src_hbm, dst_hbm, zero_hbm, out_hbm,
                    src_v, dst_v, rows_v, acc_sh, sem):
    cid = lax.axis_index("c")
    sid = lax.axis_index("s")
    wid = cid * NS + sid

    stripe = pl.ds(sid * RPW, RPW)
    tail = pl.ds(NS * RPW, RTAIL)
    pltpu.sync_copy(zero_hbm.at[stripe], acc_sh.at[stripe])

    @pl.when(sid == NS - 1)
    def _zero_tail():
        pltpu.sync_copy(zero_hbm.at[tail], acc_sh.at[tail])

    pltpu.sync_copy(src_hbm.at[wid], src_v)
    pltpu.sync_copy(dst_hbm.at[wid], dst_v)
    plsc.subcore_barrier()

    def _edge_chunk(j, carry):
        pltpu.async_copy(g_hbm.at[src_v.at[j]], rows_v, sem).wait()
        pltpu.sync_copy(rows_v, acc_sh.at[dst_v.at[j]], add=True)
        return carry

    lax.fori_loop(0, NCHUNK, _edge_chunk, 0)
    plsc.subcore_barrier()
    pltpu.sync_copy(acc_sh.at[stripe], out_hbm.at[cid].at[stripe])

    @pl.when(sid == NS - 1)
    def _dump_tail():
        pltpu.sync_copy(acc_sh.at[tail], out_hbm.at[cid].at[tail])


def _final_body(sp_ref, g_ref, dis_ref, b_ref, o_ref):
    z = (sp_ref[0] + sp_ref[1] + g_ref[...]) * dis_ref[...] + b_ref[...]
    m = jnp.max(z, axis=1, keepdims=True)
    e = z - m
    o_ref[...] = e - jnp.log(jnp.sum(jnp.exp(e), axis=1, keepdims=True))


def _tc_final(s_parts, g, dis, b2d):
    return pl.pallas_call(
        _final_body,
        grid=(N // _BR,),
        in_specs=[
            pl.BlockSpec((NC, _BR, C), lambda i: (0, i, 0)),
            pl.BlockSpec((_BR, C), lambda i: (i, 0)),
            pl.BlockSpec((_BR, 1), lambda i: (i, 0)),
            pl.BlockSpec((1, C), lambda i: (0, 0)),
        ],
        out_specs=pl.BlockSpec((_BR, C), lambda i: (i, 0)),
        out_shape=jax.ShapeDtypeStruct((N, C), jnp.float32),
    )(s_parts, g, dis, b2d)


def kernel(x, edge_index, W, b):
    src = edge_index[0].reshape(NW, NCHUNK, CHUNK)
    dst = edge_index[1]
    deg_parts = _deg_kernel(dst.reshape(NW, EPW))
    g, dis = _tc_prep(deg_parts.T, x, W)
    zeros = jnp.zeros((N, C), jnp.float32)
    s_parts = _scatter_kernel(g, src, dst.reshape(NW, NCHUNK, CHUNK), zeros)
    return _tc_final(s_parts, g, dis, b.reshape(1, C))
